# baseline (speedup 1.0000x reference)
import functools

import jax
import jax.numpy as jnp
from jax import lax
from jax.experimental import pallas as pl
from jax.experimental.pallas import tpu as pltpu
from jax.experimental.pallas import tpu_sc as plsc

_NUM_CORES = 2
_NUM_SUBCORES = 16
_NW = _NUM_CORES * _NUM_SUBCORES


@functools.lru_cache(maxsize=None)
def _build_gather(seq: int, batch: int, d_model: int):
    s_per_w = seq // _NW
    s_grp = 4
    n_grp = s_per_w // s_grp
    nbuf = 6

    mesh = plsc.VectorSubcoreMesh(
        core_axis_name="c",
        subcore_axis_name="s",
        num_cores=_NUM_CORES,
        num_subcores=_NUM_SUBCORES,
    )

    @functools.partial(
        pl.kernel,
        mesh=mesh,
        out_type=jax.ShapeDtypeStruct((seq, batch, d_model), jnp.float32),
        scratch_types=[
            pltpu.VMEM((s_per_w * batch,), jnp.int32),
            pltpu.VMEM((nbuf, s_grp * batch, d_model), jnp.float32),
            pltpu.SemaphoreType.DMA,
            pltpu.SemaphoreType.DMA,
        ],
    )
    def gather_kernel(idx_hbm, table_hbm, out_hbm, idx_v, rows_v, sem_g, sem_w):
        wid = lax.axis_index("s") * _NUM_CORES + lax.axis_index("c")
        s_base = wid * s_per_w
        pltpu.sync_copy(idx_hbm.at[wid], idx_v)

        def gather(g):
            return pltpu.async_copy(
                table_hbm.at[idx_v.at[pl.ds(g * s_grp * batch, s_grp * batch)]],
                rows_v.at[g % nbuf],
                sem_g,
            )

        def write(g):
            buf = rows_v.at[g % nbuf]
            return [
                pltpu.async_copy(
                    buf.at[pl.ds(i * batch, batch)],
                    out_hbm.at[s_base + g * s_grp + i],
                    sem_w,
                )
                for i in range(s_grp)
            ]

        h_g = [None] * n_grp
        h_w = [None] * n_grp
        for g in range(nbuf - 1):
            h_g[g] = gather(g)
        for g in range(n_grp):
            h_g[g].wait()
            h_w[g] = write(g)
            nxt = g + nbuf - 1
            if nxt < n_grp:
                if g >= 1:
                    for h in h_w[g - 1]:
                        h.wait()
                h_g[nxt] = gather(nxt)
        for g in range(n_grp - nbuf, n_grp):
            for h in h_w[g]:
                h.wait()

    return gather_kernel


def kernel(input_ids, input_mask, table):
    del input_mask
    batch, seq = input_ids.shape
    _, d_model = table.shape
    ids_t = input_ids.T.reshape(_NW, -1)
    return _build_gather(seq, batch, d_model)(ids_t, table)


# speedup vs baseline: 1.0117x; 1.0117x over previous
import functools

import jax
import jax.numpy as jnp
from jax import lax
from jax.experimental import pallas as pl
from jax.experimental.pallas import tpu as pltpu
from jax.experimental.pallas import tpu_sc as plsc

_NUM_CORES = 2
_NUM_SUBCORES = 16
_NW = _NUM_CORES * _NUM_SUBCORES


@functools.lru_cache(maxsize=None)
def _build_gather(seq: int, batch: int, d_model: int):
    s_per_w = seq // _NW
    s_grp = 8
    n_grp = (s_per_w // s_grp) * batch
    nbuf = 12

    mesh = plsc.VectorSubcoreMesh(
        core_axis_name="c",
        subcore_axis_name="s",
        num_cores=_NUM_CORES,
        num_subcores=_NUM_SUBCORES,
    )

    @functools.partial(
        pl.kernel,
        mesh=mesh,
        out_type=jax.ShapeDtypeStruct((seq, batch, d_model), jnp.float32),
        scratch_types=[
            pltpu.VMEM((batch * s_per_w,), jnp.int32),
            pltpu.VMEM((nbuf, s_grp, d_model), jnp.float32),
            pltpu.SemaphoreType.DMA,
            pltpu.SemaphoreType.DMA,
        ],
    )
    def gather_kernel(idx_hbm, table_hbm, out_hbm, idx_v, rows_v, sem_g, sem_w):
        wid = lax.axis_index("s") * _NUM_CORES + lax.axis_index("c")
        s_base = wid * s_per_w
        for b in range(batch):
            pltpu.sync_copy(
                idx_hbm.at[b, pl.ds(s_base, s_per_w)],
                idx_v.at[pl.ds(b * s_per_w, s_per_w)],
            )

        def gather(g):
            b, sblk = g % batch, g // batch
            return pltpu.async_copy(
                table_hbm.at[idx_v.at[pl.ds(b * s_per_w + sblk * s_grp, s_grp)]],
                rows_v.at[g % nbuf],
                sem_g,
            )

        def write(g):
            b, sblk = g % batch, g // batch
            return pltpu.async_copy(
                rows_v.at[g % nbuf],
                out_hbm.at[pl.ds(s_base + sblk * s_grp, s_grp), b],
                sem_w,
            )

        h_g = [None] * n_grp
        h_w = [None] * n_grp
        for g in range(nbuf - 1):
            h_g[g] = gather(g)
        for g in range(n_grp):
            h_g[g].wait()
            h_w[g] = write(g)
            nxt = g + nbuf - 1
            if nxt < n_grp:
                if g >= 1:
                    h_w[g - 1].wait()
                h_g[nxt] = gather(nxt)
        for g in range(n_grp - nbuf, n_grp):
            h_w[g].wait()

    return gather_kernel


def kernel(input_ids, input_mask, table):
    del input_mask
    batch, seq = input_ids.shape
    _, d_model = table.shape
    return _build_gather(seq, batch, d_model)(input_ids, table)


# speedup vs baseline: 1.0508x; 1.0387x over previous
import functools

import jax
import jax.numpy as jnp
from jax import lax
from jax.experimental import pallas as pl
from jax.experimental.pallas import tpu as pltpu
from jax.experimental.pallas import tpu_sc as plsc

_NUM_CORES = 2
_NUM_SUBCORES = 16
_NW = _NUM_CORES * _NUM_SUBCORES


@functools.lru_cache(maxsize=None)
def _build_gather(seq: int, batch: int, d_model: int):
    s_per_w = seq // _NW
    s_grp = 16
    n_grp = (s_per_w // s_grp) * batch
    nbuf = 7

    mesh = plsc.VectorSubcoreMesh(
        core_axis_name="c",
        subcore_axis_name="s",
        num_cores=_NUM_CORES,
        num_subcores=_NUM_SUBCORES,
    )

    @functools.partial(
        pl.kernel,
        mesh=mesh,
        out_type=jax.ShapeDtypeStruct((seq, batch, d_model), jnp.float32),
        scratch_types=[
            pltpu.VMEM((batch * s_per_w,), jnp.int32),
            pltpu.VMEM((nbuf, s_grp, d_model), jnp.float32),
            pltpu.SemaphoreType.DMA,
            pltpu.SemaphoreType.DMA,
        ],
    )
    def gather_kernel(idx_hbm, table_hbm, out_hbm, idx_v, rows_v, sem_g, sem_w):
        wid = lax.axis_index("s") * _NUM_CORES + lax.axis_index("c")
        s_base = wid * s_per_w
        for h in [
            pltpu.async_copy(
                idx_hbm.at[b, pl.ds(s_base, s_per_w)],
                idx_v.at[pl.ds(b * s_per_w, s_per_w)],
                sem_w,
            )
            for b in range(batch)
        ]:
            h.wait()

        def gather(g):
            b, sblk = g % batch, g // batch
            return pltpu.async_copy(
                table_hbm.at[idx_v.at[pl.ds(b * s_per_w + sblk * s_grp, s_grp)]],
                rows_v.at[g % nbuf],
                sem_g,
            )

        def write(g):
            b, sblk = g % batch, g // batch
            return pltpu.async_copy(
                rows_v.at[g % nbuf],
                out_hbm.at[pl.ds(s_base + sblk * s_grp, s_grp), b],
                sem_w,
            )

        h_g = [None] * n_grp
        h_w = [None] * n_grp
        for g in range(nbuf - 1):
            h_g[g] = gather(g)
        for g in range(n_grp):
            h_g[g].wait()
            h_w[g] = write(g)
            nxt = g + nbuf - 1
            if nxt < n_grp:
                if g >= 1:
                    h_w[g - 1].wait()
                h_g[nxt] = gather(nxt)
        for g in range(n_grp - nbuf, n_grp):
            h_w[g].wait()

    return gather_kernel


def kernel(input_ids, input_mask, table):
    del input_mask
    batch, seq = input_ids.shape
    _, d_model = table.shape
    return _build_gather(seq, batch, d_model)(input_ids, table)


# speedup vs baseline: 1.0586x; 1.0074x over previous
import functools

import jax
import jax.numpy as jnp
from jax import lax
from jax.experimental import pallas as pl
from jax.experimental.pallas import tpu as pltpu
from jax.experimental.pallas import tpu_sc as plsc

_NUM_CORES = 2
_NUM_SUBCORES = 16
_NW = _NUM_CORES * _NUM_SUBCORES


@functools.lru_cache(maxsize=None)
def _build_gather(seq: int, batch: int, d_model: int):
    s_per_w = seq // _NW
    s_grp = 32
    n_grp = (s_per_w // s_grp) * batch
    nbuf = 3

    mesh = plsc.VectorSubcoreMesh(
        core_axis_name="c",
        subcore_axis_name="s",
        num_cores=_NUM_CORES,
        num_subcores=_NUM_SUBCORES,
    )

    @functools.partial(
        pl.kernel,
        mesh=mesh,
        out_type=jax.ShapeDtypeStruct((seq, batch, d_model), jnp.float32),
        scratch_types=[
            pltpu.VMEM((batch * s_per_w,), jnp.int32),
            pltpu.VMEM((nbuf, s_grp, d_model), jnp.float32),
            pltpu.SemaphoreType.DMA,
            pltpu.SemaphoreType.DMA,
        ],
    )
    def gather_kernel(idx_hbm, table_hbm, out_hbm, idx_v, rows_v, sem_g, sem_w):
        wid = lax.axis_index("s") * _NUM_CORES + lax.axis_index("c")
        s_base = wid * s_per_w
        for h in [
            pltpu.async_copy(
                idx_hbm.at[b, pl.ds(s_base, s_per_w)],
                idx_v.at[pl.ds(b * s_per_w, s_per_w)],
                sem_w,
            )
            for b in range(batch)
        ]:
            h.wait()

        def gather(g):
            b, sblk = g % batch, g // batch
            return pltpu.async_copy(
                table_hbm.at[idx_v.at[pl.ds(b * s_per_w + sblk * s_grp, s_grp)]],
                rows_v.at[g % nbuf],
                sem_g,
            )

        def write(g):
            b, sblk = g % batch, g // batch
            return pltpu.async_copy(
                rows_v.at[g % nbuf],
                out_hbm.at[pl.ds(s_base + sblk * s_grp, s_grp), b],
                sem_w,
            )

        h_g = [None] * n_grp
        h_w = [None] * n_grp
        for g in range(nbuf - 1):
            h_g[g] = gather(g)
        for g in range(n_grp):
            h_g[g].wait()
            h_w[g] = write(g)
            nxt = g + nbuf - 1
            if nxt < n_grp:
                if g >= 1:
                    h_w[g - 1].wait()
                h_g[nxt] = gather(nxt)
        for g in range(n_grp - nbuf, n_grp):
            h_w[g].wait()

    return gather_kernel


def kernel(input_ids, input_mask, table):
    del input_mask
    batch, seq = input_ids.shape
    _, d_model = table.shape
    return _build_gather(seq, batch, d_model)(input_ids, table)


# speedup vs baseline: 1.0628x; 1.0040x over previous
import functools

import jax
import jax.numpy as jnp
from jax import lax
from jax.experimental import pallas as pl
from jax.experimental.pallas import tpu as pltpu
from jax.experimental.pallas import tpu_sc as plsc

_NUM_CORES = 2
_NUM_SUBCORES = 16
_NW = _NUM_CORES * _NUM_SUBCORES


@functools.lru_cache(maxsize=None)
def _build_gather(seq: int, batch: int, d_model: int):
    s_per_w = seq // _NW
    s_grp = 32
    n_grp = (s_per_w // s_grp) * batch
    nbuf = 3

    mesh = plsc.VectorSubcoreMesh(
        core_axis_name="c",
        subcore_axis_name="s",
        num_cores=_NUM_CORES,
        num_subcores=_NUM_SUBCORES,
    )

    @functools.partial(
        pl.kernel,
        mesh=mesh,
        out_type=jax.ShapeDtypeStruct((seq, batch, d_model), jnp.float32),
        scratch_types=[
            pltpu.VMEM((batch * s_per_w,), jnp.int32),
            pltpu.VMEM((nbuf, s_grp, d_model), jnp.float32),
            pltpu.SemaphoreType.DMA,
            pltpu.SemaphoreType.DMA,
        ],
    )
    def gather_kernel(idx_hbm, table_hbm, out_hbm, idx_v, rows_v, sem_g, sem_w):
        wid = lax.axis_index("s") * _NUM_CORES + lax.axis_index("c")
        s_base = wid * s_per_w
        h_i = [
            pltpu.async_copy(
                idx_hbm.at[b, pl.ds(s_base, s_per_w)],
                idx_v.at[pl.ds(b * s_per_w, s_per_w)],
                sem_w,
            )
            for b in range(batch)
        ]

        def gather(g):
            b, sblk = g % batch, g // batch
            return pltpu.async_copy(
                table_hbm.at[idx_v.at[pl.ds(b * s_per_w + sblk * s_grp, s_grp)]],
                rows_v.at[g % nbuf],
                sem_g,
            )

        def write(g):
            b, sblk = g % batch, g // batch
            return pltpu.async_copy(
                rows_v.at[g % nbuf],
                out_hbm.at[pl.ds(s_base + sblk * s_grp, s_grp), b],
                sem_w,
            )

        h_g = [None] * n_grp
        h_w = [None] * n_grp
        for g in range(nbuf - 1):
            h_i[g % batch].wait()
            h_g[g] = gather(g)
        for b in range(min(nbuf - 1, batch), batch):
            h_i[b].wait()
        for g in range(n_grp):
            h_g[g].wait()
            h_w[g] = write(g)
            nxt = g + nbuf - 1
            if nxt < n_grp:
                if g >= 1:
                    h_w[g - 1].wait()
                h_g[nxt] = gather(nxt)
        for g in range(n_grp - nbuf, n_grp):
            h_w[g].wait()

    return gather_kernel


def kernel(input_ids, input_mask, table):
    del input_mask
    batch, seq = input_ids.shape
    _, d_model = table.shape
    return _build_gather(seq, batch, d_model)(input_ids, table)
